# baseline (speedup 1.0000x reference)
import functools

import jax
import jax.numpy as jnp
from jax.experimental import pallas as pl
from jax.experimental.pallas import tpu as pltpu

F32 = jnp.float32
BF16 = jnp.bfloat16
BIG = 3.0e38


def _dot(a, b):
    return jax.lax.dot_general(a, b, (((1,), (0,)), ((), ())),
                               preferred_element_type=F32,
                               precision=jax.lax.Precision.DEFAULT)


def _dott(a, b):
    return jax.lax.dot_general(a, b, (((0,), (0,)), ((), ())),
                               preferred_element_type=F32,
                               precision=jax.lax.Precision.HIGHEST)


def _p1_body(K, Q, N, keys_ref, kt_ref, k24_ref, concat_ref, mom_ref,
             d2_scr):
    b = pl.program_id(0)
    qi = pl.program_id(1)
    k24 = k24_ref[0]
    kt = kt_ref[0]
    q = keys_ref[0, pl.ds(qi * Q, Q), :]
    q3 = q[:, 0:3]
    sqq = jnp.sum(q * q, axis=1, keepdims=True)
    sqk = jnp.sum(kt * kt, axis=0, keepdims=True)
    gram = jax.lax.dot_general(q, kt, (((1,), (0,)), ((), ())),
                               preferred_element_type=F32,
                               precision=jax.lax.Precision.DEFAULT)
    d2_scr[...] = jnp.maximum(sqq + sqk - 2.0 * gram, 0.0)
    iota = jax.lax.broadcasted_iota(jnp.int32, (Q, N), 1)

    m_prev = jnp.full((Q, 1), -1.0, F32)
    sel_prev = jnp.full((Q, 1), N, jnp.int32)
    d2v = d2_scr[...]
    for k in range(K):
        cand = jnp.logical_or(
            d2v > m_prev,
            jnp.logical_and(d2v == m_prev, iota > sel_prev))
        m = jnp.min(jnp.where(cand, d2v, BIG), axis=1, keepdims=True)
        sel = jnp.min(jnp.where(jnp.logical_and(cand, d2v == m), iota, N),
                      axis=1, keepdims=True)
        ohb = (iota == sel).astype(BF16)
        nb24 = _dot(ohb, k24)
        nb = (nb24[:, 0:3] + nb24[:, 8:11]) + nb24[:, 16:19]
        row = jnp.concatenate(
            [q3, nb, q3 - nb, jnp.sqrt(m + 1e-12),
             jnp.zeros((Q, 6), F32)], axis=1)
        concat_ref[0, k:k + 1] = row[None]
        m_prev, sel_prev = m, sel
    flat = jnp.reshape(concat_ref[0], (K * Q, 16))
    m2 = _dott(flat, flat)
    sc = jnp.sum(flat, axis=0, keepdims=True)

    @pl.when(jnp.logical_and(b == 0, qi == 0))
    def _():
        mom_ref[...] = jnp.zeros_like(mom_ref)

    mom_ref[0:16, :] += m2
    mom_ref[16:17, :] += sc


def _p2_body(K, Q, concat_ref, feat_ref, w1_ref, b1_ref, w2_ref, b2_ref,
             sw1_ref, sw2_ref, m1w_ref, m1b_ref, o1g_ref, o1f_ref, o1b_ref,
             shw_ref, shb_ref,
             g2_ref, y1_ref, ys_ref, st1_ref, sts_ref):
    b = pl.program_id(0)
    qi = pl.program_id(1)
    flat = jnp.reshape(concat_ref[0], (K * Q, 16))
    enc1 = jnp.maximum(_dot(flat, w1_ref[...]) + b1_ref[...], 0.0)
    enc2 = jnp.maximum(_dot(flat, w2_ref[...]) + b2_ref[...], 0.0)

    def pool(enc, sw):
        a = _dot(enc, sw)
        ar = jnp.reshape(a, (K, Q, 64))
        er = jnp.reshape(enc, (K, Q, 64))
        mx = jnp.max(ar, axis=0, keepdims=True)
        ex = jnp.exp(ar - mx)
        s = ex / jnp.sum(ex, axis=0, keepdims=True)
        return jnp.sum(s * er, axis=0)

    g1 = pool(enc1, sw1_ref[...])
    g2 = pool(enc2, sw2_ref[...])
    g2_ref[0] = g2
    f = feat_ref[0]
    x0 = _dot(f, m1w_ref[...]) + m1b_ref[...]
    x0 = jnp.where(x0 >= 0, x0, 0.2 * x0)
    y1 = _dot(g1, o1g_ref[...]) + _dot(x0, o1f_ref[...]) + o1b_ref[...]
    y1_ref[0] = y1
    ys = _dot(f, shw_ref[...]) + shb_ref[...]
    ys_ref[0] = ys

    @pl.when(jnp.logical_and(b == 0, qi == 0))
    def _():
        st1_ref[...] = jnp.zeros_like(st1_ref)
        sts_ref[...] = jnp.zeros_like(sts_ref)

    st1_ref[0:1, :] += jnp.sum(y1, axis=0, keepdims=True)
    st1_ref[1:2, :] += jnp.sum(y1 * y1, axis=0, keepdims=True)
    sts_ref[0:1, :] += jnp.sum(ys, axis=0, keepdims=True)
    sts_ref[1:2, :] += jnp.sum(ys * ys, axis=0, keepdims=True)


def _p3_body(y1_ref, g2_ref, o2g_ref, o2f_ref, o2b_ref, sc1_ref, sh1_ref,
             y2_ref, st2_ref):
    b = pl.program_id(0)
    qi = pl.program_id(1)
    x1 = jnp.maximum(y1_ref[0] * sc1_ref[...] + sh1_ref[...], 0.0)
    y2 = _dot(g2_ref[0], o2g_ref[...]) + _dot(x1, o2f_ref[...]) + o2b_ref[...]
    y2_ref[0] = y2

    @pl.when(jnp.logical_and(b == 0, qi == 0))
    def _():
        st2_ref[...] = jnp.zeros_like(st2_ref)

    st2_ref[0:1, :] += jnp.sum(y2, axis=0, keepdims=True)
    st2_ref[1:2, :] += jnp.sum(y2 * y2, axis=0, keepdims=True)


def _p4_body(y2_ref, ys_ref, m2w_ref, m2b_ref, sc2_ref, sh2_ref, scs_ref,
             shs_ref, out_ref):
    x2 = jnp.maximum(y2_ref[0] * sc2_ref[...] + sh2_ref[...], 0.0)
    o = _dot(x2, m2w_ref[...]) + m2b_ref[...] + ys_ref[0] * scs_ref[...] \
        + shs_ref[...]
    out_ref[0] = jnp.where(o >= 0, o, 0.01 * o)


def kernel(coords, features, mlp1_w, mlp1_b, lse1_w, lse1_b, lse1_g, lse1_bt,
           pool1_sw, pool1_sb, pool1_ow, pool1_ob, pool1_g, pool1_bt,
           lse2_w, lse2_b, lse2_g, lse2_bt, pool2_sw, pool2_sb, pool2_ow,
           pool2_ob, pool2_g, pool2_bt, mlp2_w, mlp2_b, short_w, short_b,
           short_g, short_bt):
    B, N, _ = coords.shape
    K = 16
    h = mlp1_w.shape[0]
    d_in = mlp1_w.shape[1]
    d_out = pool1_sw.shape[0]
    d_fin = mlp2_w.shape[0]
    eps = 1e-6

    coords_pad = jnp.concatenate(
        [coords, jnp.zeros((B, N, 8 - coords.shape[2]), F32)], axis=2)
    coords_t = jnp.transpose(coords_pad, (0, 2, 1))
    feat_t = jnp.transpose(features[:, :, :, 0], (0, 2, 1))
    k_hi = coords_pad.astype(BF16)
    r1 = coords_pad - k_hi.astype(F32)
    k_mid = r1.astype(BF16)
    k_lo = (r1 - k_mid.astype(F32)).astype(BF16)
    k24 = jnp.concatenate([k_hi, k_mid, k_lo], axis=2)

    Q1 = 512
    nb1 = N // Q1
    p1 = pl.pallas_call(
        functools.partial(_p1_body, K, Q1, N),
        grid=(B, nb1),
        in_specs=[
            pl.BlockSpec((1, N, 8), lambda b, q: (b, 0, 0)),
            pl.BlockSpec((1, 8, N), lambda b, q: (b, 0, 0)),
            pl.BlockSpec((1, N, 24), lambda b, q: (b, 0, 0)),
        ],
        out_specs=[
            pl.BlockSpec((1, K, Q1, 16), lambda b, q: (b, 0, q, 0)),
            pl.BlockSpec((24, 16), lambda b, q: (0, 0)),
        ],
        out_shape=[
            jax.ShapeDtypeStruct((B, K, N, 16), F32),
            jax.ShapeDtypeStruct((24, 16), F32),
        ],
        scratch_shapes=[
            pltpu.VMEM((Q1, N), F32),
        ],
    )
    concat, mom = p1(coords_pad, coords_t, k24)

    M = B * N * K
    mu = mom[16, 0:10] / M
    m2 = mom[0:10, 0:10] / M
    cc = m2 - jnp.outer(mu, mu)

    def fold(w, bb, g, bt):
        mean = w @ mu + bb
        var = jnp.sum((w @ cc) * w, axis=1)
        sc = g / jnp.sqrt(var + eps)
        weff = jnp.zeros((16, h), F32).at[0:10, :].set((w * sc[:, None]).T)
        beff = (bb - mean) * sc + bt
        return weff, beff[None, :]

    w1eff, b1eff = fold(lse1_w, lse1_b, lse1_g, lse1_bt)
    w2eff, b2eff = fold(lse2_w, lse2_b, lse2_g, lse2_bt)

    Q2 = 512
    nb2 = N // Q2
    cst = lambda b, q: (0, 0)
    p2 = pl.pallas_call(
        functools.partial(_p2_body, K, Q2),
        grid=(B, nb2),
        in_specs=[
            pl.BlockSpec((1, K, Q2, 16), lambda b, q: (b, 0, q, 0)),
            pl.BlockSpec((1, Q2, d_in), lambda b, q: (b, q, 0)),
            pl.BlockSpec((16, h), cst), pl.BlockSpec((1, h), cst),
            pl.BlockSpec((16, h), cst), pl.BlockSpec((1, h), cst),
            pl.BlockSpec((h, h), cst), pl.BlockSpec((h, h), cst),
            pl.BlockSpec((d_in, h), cst), pl.BlockSpec((1, h), cst),
            pl.BlockSpec((h, h), cst), pl.BlockSpec((h, h), cst),
            pl.BlockSpec((1, h), cst),
            pl.BlockSpec((d_in, d_fin), cst), pl.BlockSpec((1, d_fin), cst),
        ],
        out_specs=[
            pl.BlockSpec((1, Q2, h), lambda b, q: (b, q, 0)),
            pl.BlockSpec((1, Q2, h), lambda b, q: (b, q, 0)),
            pl.BlockSpec((1, Q2, d_fin), lambda b, q: (b, q, 0)),
            pl.BlockSpec((8, h), cst),
            pl.BlockSpec((8, d_fin), cst),
        ],
        out_shape=[
            jax.ShapeDtypeStruct((B, N, h), F32),
            jax.ShapeDtypeStruct((B, N, h), F32),
            jax.ShapeDtypeStruct((B, N, d_fin), F32),
            jax.ShapeDtypeStruct((8, h), F32),
            jax.ShapeDtypeStruct((8, d_fin), F32),
        ],
    )
    g2, y1, ys, st1, sts = p2(
        concat, feat_t,
        w1eff, b1eff, w2eff, b2eff,
        pool1_sw[0:h, 0:h].T, pool2_sw[0:h, 0:h].T,
        mlp1_w.T, mlp1_b[None, :],
        pool1_ow[:, 0:h].T, pool1_ow[:, h:].T, pool1_ob[None, :],
        short_w.T, short_b[None, :])

    def bnstats(st, g, bt, cnt):
        mean = st[0] / cnt
        var = st[1] / cnt - mean * mean
        sc = g / jnp.sqrt(var + eps)
        return sc[None, :], (bt - mean * sc)[None, :]

    sc1, sh1 = bnstats(st1, pool1_g, pool1_bt, B * N)
    scs, shs = bnstats(sts, short_g, short_bt, B * N)

    Q3 = 2048
    nb3 = N // Q3
    p3 = pl.pallas_call(
        _p3_body,
        grid=(B, nb3),
        in_specs=[
            pl.BlockSpec((1, Q3, h), lambda b, q: (b, q, 0)),
            pl.BlockSpec((1, Q3, h), lambda b, q: (b, q, 0)),
            pl.BlockSpec((h, d_out), cst), pl.BlockSpec((h, d_out), cst),
            pl.BlockSpec((1, d_out), cst),
            pl.BlockSpec((1, h), cst), pl.BlockSpec((1, h), cst),
        ],
        out_specs=[
            pl.BlockSpec((1, Q3, d_out), lambda b, q: (b, q, 0)),
            pl.BlockSpec((8, d_out), cst),
        ],
        out_shape=[
            jax.ShapeDtypeStruct((B, N, d_out), F32),
            jax.ShapeDtypeStruct((8, d_out), F32),
        ],
    )
    y2, st2 = p3(y1, g2, pool2_ow[:, 0:h].T, pool2_ow[:, h:].T,
                 pool2_ob[None, :], sc1, sh1)

    sc2, sh2 = bnstats(st2, pool2_g, pool2_bt, B * N)

    p4 = pl.pallas_call(
        _p4_body,
        grid=(B, nb3),
        in_specs=[
            pl.BlockSpec((1, Q3, d_out), lambda b, q: (b, q, 0)),
            pl.BlockSpec((1, Q3, d_fin), lambda b, q: (b, q, 0)),
            pl.BlockSpec((d_out, d_fin), cst), pl.BlockSpec((1, d_fin), cst),
            pl.BlockSpec((1, d_out), cst), pl.BlockSpec((1, d_out), cst),
            pl.BlockSpec((1, d_fin), cst), pl.BlockSpec((1, d_fin), cst),
        ],
        out_specs=[pl.BlockSpec((1, Q3, d_fin), lambda b, q: (b, q, 0))],
        out_shape=[jax.ShapeDtypeStruct((B, N, d_fin), F32)],
    )
    (out,) = p4(y2, ys, mlp2_w.T, mlp2_b[None, :], sc2, sh2, scs, shs)

    return jnp.transpose(out, (0, 2, 1))[:, :, :, None]


# speedup vs baseline: 1.0279x; 1.0279x over previous
import functools

import jax
import jax.numpy as jnp
from jax import lax
from jax.experimental import pallas as pl
from jax.experimental.pallas import tpu as pltpu
from jax.experimental.pallas import tpu_sc as plsc

F32 = jnp.float32
BF16 = jnp.bfloat16
BIG = 3.0e38


def _dot(a, b):
    return jax.lax.dot_general(a, b, (((1,), (0,)), ((), ())),
                               preferred_element_type=F32,
                               precision=jax.lax.Precision.DEFAULT)


def _dott(a, b):
    return jax.lax.dot_general(a, b, (((0,), (0,)), ((), ())),
                               preferred_element_type=F32,
                               precision=jax.lax.Precision.HIGHEST)


def _p1_body(K, Q, N, keys_ref, kt_ref, idx_ref, dist_ref, d2_scr):
    b = pl.program_id(0)
    qi = pl.program_id(1)
    kt = kt_ref[0]
    q = keys_ref[0, pl.ds(qi * Q, Q), :]
    sqq = jnp.sum(q * q, axis=1, keepdims=True)
    sqk = jnp.sum(kt * kt, axis=0, keepdims=True)
    gram = jax.lax.dot_general(q, kt, (((1,), (0,)), ((), ())),
                               preferred_element_type=F32,
                               precision=jax.lax.Precision.DEFAULT)
    d2_scr[...] = jnp.maximum(sqq + sqk - 2.0 * gram, 0.0)
    iota = jax.lax.broadcasted_iota(jnp.int32, (Q, N), 1)

    m_prev = jnp.full((Q, 1), -1.0, F32)
    sel_prev = jnp.full((Q, 1), N, jnp.int32)
    d2v = d2_scr[...]
    for k in range(K):
        cand = jnp.logical_or(
            d2v > m_prev,
            jnp.logical_and(d2v == m_prev, iota > sel_prev))
        m = jnp.min(jnp.where(cand, d2v, BIG), axis=1, keepdims=True)
        sel = jnp.min(jnp.where(jnp.logical_and(cand, d2v == m), iota, N),
                      axis=1, keepdims=True)
        idx_ref[0, :, k:k + 1] = sel + b * N
        dist_ref[0, :, k:k + 1] = jnp.sqrt(m + 1e-12)
        m_prev, sel_prev = m, sel


def _make_sc_gather(n_rows, n_idx):
    info = plsc.get_sparse_core_info()
    nc, ns = info.num_cores, info.num_subcores
    nw = nc * ns
    per_w = n_idx // nw
    mesh = plsc.VectorSubcoreMesh(core_axis_name="c", subcore_axis_name="s")

    @functools.partial(
        pl.kernel, mesh=mesh,
        out_type=jax.ShapeDtypeStruct((n_idx * 8,), F32),
        compiler_params=pltpu.CompilerParams(needs_layout_passes=False),
        scratch_types=[
            pltpu.VMEM((n_rows * 8,), F32),
            pltpu.VMEM((per_w,), jnp.int32),
            pltpu.VMEM((per_w * 8,), F32),
        ],
    )
    def g(table_hbm, idx_hbm, out_hbm, table_v, idx_v, out_v):
        wid = lax.axis_index("s") * nc + lax.axis_index("c")
        base = wid * per_w
        pltpu.sync_copy(table_hbm, table_v)
        pltpu.sync_copy(idx_hbm.at[pl.ds(base, per_w)], idx_v)
        lane = lax.iota(jnp.int32, 16)

        def body(gi, carry):
            idx16 = idx_v[pl.ds(gi * 16, 16)]
            rowbase = idx16 * 8
            obase = lane * 8 + gi * 128
            for c in range(3):
                vals = plsc.load_gather(table_v, [rowbase + c])
                plsc.store_scatter(out_v, [obase + c], vals)
            return carry

        jax.lax.fori_loop(0, per_w // 16, body, 0)
        pltpu.sync_copy(out_v, out_hbm.at[pl.ds(base * 8, per_w * 8)])

    return g


def _build_cat(K, Q, ctr, nbv, dd):
    c3 = jnp.broadcast_to(jnp.reshape(ctr[:, 0:3], (Q, 1, 3)), (Q, K, 3))
    nb3 = nbv[:, :, 0:3]
    cat = jnp.concatenate(
        [c3, nb3, c3 - nb3, jnp.reshape(dd, (Q, K, 1)),
         jnp.zeros((Q, K, 6), F32)], axis=2)
    return jnp.reshape(cat, (Q * K, 16))


def _p2a_body(K, Q, nb_ref, ctr_ref, dist_ref, mom_ref):
    b = pl.program_id(0)
    qi = pl.program_id(1)
    flat = _build_cat(K, Q, ctr_ref[0], nb_ref[0], dist_ref[0])

    @pl.when(jnp.logical_and(b == 0, qi == 0))
    def _():
        mom_ref[...] = jnp.zeros_like(mom_ref)

    mom_ref[0:16, :] += _dott(flat, flat)
    mom_ref[16:17, :] += jnp.sum(flat, axis=0, keepdims=True)


def _p2b_body(K, Q, nb_ref, ctr_ref, dist_ref, feat_ref, w1_ref, b1_ref,
              w2_ref, b2_ref, sw1_ref, sw2_ref, m1w_ref, m1b_ref, o1g_ref,
              o1f_ref, o1b_ref, shw_ref, shb_ref,
              g2_ref, y1_ref, ys_ref, st1_ref, sts_ref):
    b = pl.program_id(0)
    qi = pl.program_id(1)
    flat = _build_cat(K, Q, ctr_ref[0], nb_ref[0], dist_ref[0])
    enc1 = jnp.maximum(_dot(flat, w1_ref[...]) + b1_ref[...], 0.0)
    enc2 = jnp.maximum(_dot(flat, w2_ref[...]) + b2_ref[...], 0.0)

    def pool(enc, sw):
        a = _dot(enc, sw)
        ar = jnp.reshape(a, (Q, K, 64))
        er = jnp.reshape(enc, (Q, K, 64))
        mx = jnp.max(ar, axis=1, keepdims=True)
        ex = jnp.exp(ar - mx)
        s = ex / jnp.sum(ex, axis=1, keepdims=True)
        return jnp.sum(s * er, axis=1)

    g1 = pool(enc1, sw1_ref[...])
    g2 = pool(enc2, sw2_ref[...])
    g2_ref[0] = g2
    f = feat_ref[0]
    x0 = _dot(f, m1w_ref[...]) + m1b_ref[...]
    x0 = jnp.where(x0 >= 0, x0, 0.2 * x0)
    y1 = _dot(g1, o1g_ref[...]) + _dot(x0, o1f_ref[...]) + o1b_ref[...]
    y1_ref[0] = y1
    ys = _dot(f, shw_ref[...]) + shb_ref[...]
    ys_ref[0] = ys

    @pl.when(jnp.logical_and(b == 0, qi == 0))
    def _():
        st1_ref[...] = jnp.zeros_like(st1_ref)
        sts_ref[...] = jnp.zeros_like(sts_ref)

    st1_ref[0:1, :] += jnp.sum(y1, axis=0, keepdims=True)
    st1_ref[1:2, :] += jnp.sum(y1 * y1, axis=0, keepdims=True)
    sts_ref[0:1, :] += jnp.sum(ys, axis=0, keepdims=True)
    sts_ref[1:2, :] += jnp.sum(ys * ys, axis=0, keepdims=True)


def _p3_body(y1_ref, g2_ref, o2g_ref, o2f_ref, o2b_ref, sc1_ref, sh1_ref,
             y2_ref, st2_ref):
    b = pl.program_id(0)
    qi = pl.program_id(1)
    x1 = jnp.maximum(y1_ref[0] * sc1_ref[...] + sh1_ref[...], 0.0)
    y2 = _dot(g2_ref[0], o2g_ref[...]) + _dot(x1, o2f_ref[...]) + o2b_ref[...]
    y2_ref[0] = y2

    @pl.when(jnp.logical_and(b == 0, qi == 0))
    def _():
        st2_ref[...] = jnp.zeros_like(st2_ref)

    st2_ref[0:1, :] += jnp.sum(y2, axis=0, keepdims=True)
    st2_ref[1:2, :] += jnp.sum(y2 * y2, axis=0, keepdims=True)


def _p4_body(y2_ref, ys_ref, m2w_ref, m2b_ref, sc2_ref, sh2_ref, scs_ref,
             shs_ref, out_ref):
    x2 = jnp.maximum(y2_ref[0] * sc2_ref[...] + sh2_ref[...], 0.0)
    o = _dot(x2, m2w_ref[...]) + m2b_ref[...] + ys_ref[0] * scs_ref[...] \
        + shs_ref[...]
    out_ref[0] = jnp.where(o >= 0, o, 0.01 * o)


def kernel(coords, features, mlp1_w, mlp1_b, lse1_w, lse1_b, lse1_g, lse1_bt,
           pool1_sw, pool1_sb, pool1_ow, pool1_ob, pool1_g, pool1_bt,
           lse2_w, lse2_b, lse2_g, lse2_bt, pool2_sw, pool2_sb, pool2_ow,
           pool2_ob, pool2_g, pool2_bt, mlp2_w, mlp2_b, short_w, short_b,
           short_g, short_bt):
    B, N, _ = coords.shape
    K = 16
    h = mlp1_w.shape[0]
    d_in = mlp1_w.shape[1]
    d_out = pool1_sw.shape[0]
    d_fin = mlp2_w.shape[0]
    eps = 1e-6

    coords_pad = jnp.concatenate(
        [coords, jnp.zeros((B, N, 8 - coords.shape[2]), F32)], axis=2)
    coords_t = jnp.transpose(coords_pad, (0, 2, 1))
    feat_t = jnp.transpose(features[:, :, :, 0], (0, 2, 1))

    Q1 = 512
    nb1 = N // Q1
    p1 = pl.pallas_call(
        functools.partial(_p1_body, K, Q1, N),
        grid=(B, nb1),
        in_specs=[
            pl.BlockSpec((1, N, 8), lambda b, q: (b, 0, 0)),
            pl.BlockSpec((1, 8, N), lambda b, q: (b, 0, 0)),
        ],
        out_specs=[
            pl.BlockSpec((1, Q1, K), lambda b, q: (b, q, 0)),
            pl.BlockSpec((1, Q1, K), lambda b, q: (b, q, 0)),
        ],
        out_shape=[
            jax.ShapeDtypeStruct((B, N, K), jnp.int32),
            jax.ShapeDtypeStruct((B, N, K), F32),
        ],
        scratch_shapes=[pltpu.VMEM((Q1, N), F32)],
    )
    idx, dist = p1(coords_pad, coords_t)

    table = jnp.reshape(coords_pad, (B * N * 8,))
    gfn = _make_sc_gather(B * N, B * N * K)
    nb_flat = gfn(table, jnp.reshape(idx, (B * N * K,)))
    nbg = jnp.reshape(nb_flat, (B, N, K, 8))

    Q2 = 512
    nb2 = N // Q2
    cst = lambda b, q: (0, 0)
    p2a = pl.pallas_call(
        functools.partial(_p2a_body, K, Q2),
        grid=(B, nb2),
        in_specs=[
            pl.BlockSpec((1, Q2, K, 8), lambda b, q: (b, q, 0, 0)),
            pl.BlockSpec((1, Q2, 8), lambda b, q: (b, q, 0)),
            pl.BlockSpec((1, Q2, K), lambda b, q: (b, q, 0)),
        ],
        out_specs=[pl.BlockSpec((24, 16), cst)],
        out_shape=[jax.ShapeDtypeStruct((24, 16), F32)],
    )
    (mom,) = p2a(nbg, coords_pad, dist)

    M = B * N * K
    mu = mom[16, 0:10] / M
    m2 = mom[0:10, 0:10] / M
    cc = m2 - jnp.outer(mu, mu)

    def fold(w, bb, g, bt):
        mean = w @ mu + bb
        var = jnp.sum((w @ cc) * w, axis=1)
        sc = g / jnp.sqrt(var + eps)
        weff = jnp.zeros((16, h), F32).at[0:10, :].set((w * sc[:, None]).T)
        beff = (bb - mean) * sc + bt
        return weff, beff[None, :]

    w1eff, b1eff = fold(lse1_w, lse1_b, lse1_g, lse1_bt)
    w2eff, b2eff = fold(lse2_w, lse2_b, lse2_g, lse2_bt)

    p2b = pl.pallas_call(
        functools.partial(_p2b_body, K, Q2),
        grid=(B, nb2),
        in_specs=[
            pl.BlockSpec((1, Q2, K, 8), lambda b, q: (b, q, 0, 0)),
            pl.BlockSpec((1, Q2, 8), lambda b, q: (b, q, 0)),
            pl.BlockSpec((1, Q2, K), lambda b, q: (b, q, 0)),
            pl.BlockSpec((1, Q2, d_in), lambda b, q: (b, q, 0)),
            pl.BlockSpec((16, h), cst), pl.BlockSpec((1, h), cst),
            pl.BlockSpec((16, h), cst), pl.BlockSpec((1, h), cst),
            pl.BlockSpec((h, h), cst), pl.BlockSpec((h, h), cst),
            pl.BlockSpec((d_in, h), cst), pl.BlockSpec((1, h), cst),
            pl.BlockSpec((h, h), cst), pl.BlockSpec((h, h), cst),
            pl.BlockSpec((1, h), cst),
            pl.BlockSpec((d_in, d_fin), cst), pl.BlockSpec((1, d_fin), cst),
        ],
        out_specs=[
            pl.BlockSpec((1, Q2, h), lambda b, q: (b, q, 0)),
            pl.BlockSpec((1, Q2, h), lambda b, q: (b, q, 0)),
            pl.BlockSpec((1, Q2, d_fin), lambda b, q: (b, q, 0)),
            pl.BlockSpec((8, h), cst),
            pl.BlockSpec((8, d_fin), cst),
        ],
        out_shape=[
            jax.ShapeDtypeStruct((B, N, h), F32),
            jax.ShapeDtypeStruct((B, N, h), F32),
            jax.ShapeDtypeStruct((B, N, d_fin), F32),
            jax.ShapeDtypeStruct((8, h), F32),
            jax.ShapeDtypeStruct((8, d_fin), F32),
        ],
    )
    g2, y1, ys, st1, sts = p2b(
        nbg, coords_pad, dist, feat_t,
        w1eff, b1eff, w2eff, b2eff,
        pool1_sw[0:h, 0:h].T, pool2_sw[0:h, 0:h].T,
        mlp1_w.T, mlp1_b[None, :],
        pool1_ow[:, 0:h].T, pool1_ow[:, h:].T, pool1_ob[None, :],
        short_w.T, short_b[None, :])

    def bnstats(st, g, bt, cnt):
        mean = st[0] / cnt
        var = st[1] / cnt - mean * mean
        sc = g / jnp.sqrt(var + eps)
        return sc[None, :], (bt - mean * sc)[None, :]

    sc1, sh1 = bnstats(st1, pool1_g, pool1_bt, B * N)
    scs, shs = bnstats(sts, short_g, short_bt, B * N)

    Q3 = 2048
    nb3 = N // Q3
    p3 = pl.pallas_call(
        _p3_body,
        grid=(B, nb3),
        in_specs=[
            pl.BlockSpec((1, Q3, h), lambda b, q: (b, q, 0)),
            pl.BlockSpec((1, Q3, h), lambda b, q: (b, q, 0)),
            pl.BlockSpec((h, d_out), cst), pl.BlockSpec((h, d_out), cst),
            pl.BlockSpec((1, d_out), cst),
            pl.BlockSpec((1, h), cst), pl.BlockSpec((1, h), cst),
        ],
        out_specs=[
            pl.BlockSpec((1, Q3, d_out), lambda b, q: (b, q, 0)),
            pl.BlockSpec((8, d_out), cst),
        ],
        out_shape=[
            jax.ShapeDtypeStruct((B, N, d_out), F32),
            jax.ShapeDtypeStruct((8, d_out), F32),
        ],
    )
    y2, st2 = p3(y1, g2, pool2_ow[:, 0:h].T, pool2_ow[:, h:].T,
                 pool2_ob[None, :], sc1, sh1)

    sc2, sh2 = bnstats(st2, pool2_g, pool2_bt, B * N)

    p4 = pl.pallas_call(
        _p4_body,
        grid=(B, nb3),
        in_specs=[
            pl.BlockSpec((1, Q3, d_out), lambda b, q: (b, q, 0)),
            pl.BlockSpec((1, Q3, d_fin), lambda b, q: (b, q, 0)),
            pl.BlockSpec((d_out, d_fin), cst), pl.BlockSpec((1, d_fin), cst),
            pl.BlockSpec((1, d_out), cst), pl.BlockSpec((1, d_out), cst),
            pl.BlockSpec((1, d_fin), cst), pl.BlockSpec((1, d_fin), cst),
        ],
        out_specs=[pl.BlockSpec((1, Q3, d_fin), lambda b, q: (b, q, 0))],
        out_shape=[jax.ShapeDtypeStruct((B, N, d_fin), F32)],
    )
    (out,) = p4(y2, ys, mlp2_w.T, mlp2_b[None, :], sc2, sh2, scs, shs)

    return jnp.transpose(out, (0, 2, 1))[:, :, :, None]


# speedup vs baseline: 1.6282x; 1.5840x over previous
import functools

import jax
import jax.numpy as jnp
from jax import lax
from jax.experimental import pallas as pl
from jax.experimental.pallas import tpu as pltpu
from jax.experimental.pallas import tpu_sc as plsc

F32 = jnp.float32
BF16 = jnp.bfloat16
BIG = 3.0e38


def _dot(a, b):
    return jax.lax.dot_general(a, b, (((1,), (0,)), ((), ())),
                               preferred_element_type=F32,
                               precision=jax.lax.Precision.DEFAULT)


def _dott(a, b):
    return jax.lax.dot_general(a, b, (((0,), (0,)), ((), ())),
                               preferred_element_type=F32,
                               precision=jax.lax.Precision.HIGHEST)


def _p1_body(K, Q, N, keys_ref, kt_ref, idx_ref, dist_ref, d2_scr):
    b = pl.program_id(0)
    qi = pl.program_id(1)
    kt = kt_ref[0]
    q = keys_ref[0, pl.ds(qi * Q, Q), :]
    sqq = jnp.sum(q * q, axis=1, keepdims=True)
    sqk = jnp.sum(kt * kt, axis=0, keepdims=True)
    gram = jax.lax.dot_general(q, kt, (((1,), (0,)), ((), ())),
                               preferred_element_type=F32,
                               precision=jax.lax.Precision.DEFAULT)
    d2_scr[...] = jnp.maximum(sqq + sqk - 2.0 * gram, 0.0)
    iota = jax.lax.broadcasted_iota(jnp.int32, (Q, N), 1)

    m = jnp.min(d2_scr[...], axis=1, keepdims=True)
    for k in range(K):
        d2v = d2_scr[...]
        sel = jnp.min(jnp.where(d2v == m, iota, N), axis=1,
                      keepdims=True)
        idx_ref[0, :, k:k + 1] = sel + b * N
        dist_ref[0, :, k:k + 1] = jnp.sqrt(m + 1e-12)
        d2n = jnp.where(iota == sel, BIG, d2v)
        d2_scr[...] = d2n
        m = jnp.min(d2n, axis=1, keepdims=True)


def _make_sc_gather(n_rows, n_idx):
    info = plsc.get_sparse_core_info()
    nc, ns = info.num_cores, info.num_subcores
    nw = nc * ns
    per_w = n_idx // nw
    mesh = plsc.VectorSubcoreMesh(core_axis_name="c", subcore_axis_name="s")

    @functools.partial(
        pl.kernel, mesh=mesh,
        out_type=jax.ShapeDtypeStruct((n_idx * 8,), F32),
        compiler_params=pltpu.CompilerParams(needs_layout_passes=False),
        scratch_types=[
            pltpu.VMEM((n_rows * 8,), F32),
            pltpu.VMEM((per_w,), jnp.int32),
            pltpu.VMEM((per_w * 8,), F32),
        ],
    )
    def g(table_hbm, idx_hbm, out_hbm, table_v, idx_v, out_v):
        wid = lax.axis_index("s") * nc + lax.axis_index("c")
        base = wid * per_w
        pltpu.sync_copy(table_hbm, table_v)
        pltpu.sync_copy(idx_hbm.at[pl.ds(base, per_w)], idx_v)
        lane = lax.iota(jnp.int32, 16)

        def body(gi, carry):
            idx16 = idx_v[pl.ds(gi * 16, 16)]
            rowbase = idx16 * 8
            obase = lane * 8 + gi * 128
            for c in range(3):
                vals = plsc.load_gather(table_v, [rowbase + c])
                plsc.store_scatter(out_v, [obase + c], vals)
            return carry

        jax.lax.fori_loop(0, per_w // 16, body, 0)
        pltpu.sync_copy(out_v, out_hbm.at[pl.ds(base * 8, per_w * 8)])

    return g


def _build_cat(K, Q, ctr, nbv, dd):
    c3 = jnp.broadcast_to(jnp.reshape(ctr[:, 0:3], (Q, 1, 3)), (Q, K, 3))
    nb3 = nbv[:, :, 0:3]
    cat = jnp.concatenate(
        [c3, nb3, c3 - nb3, jnp.reshape(dd, (Q, K, 1)),
         jnp.zeros((Q, K, 6), F32)], axis=2)
    return jnp.reshape(cat, (Q * K, 16))


def _p2a_body(K, Q, nb_ref, ctr_ref, dist_ref, mom_ref):
    b = pl.program_id(0)
    qi = pl.program_id(1)
    flat = _build_cat(K, Q, ctr_ref[0], nb_ref[0], dist_ref[0])

    @pl.when(jnp.logical_and(b == 0, qi == 0))
    def _():
        mom_ref[...] = jnp.zeros_like(mom_ref)

    mom_ref[0:16, :] += _dott(flat, flat)
    mom_ref[16:17, :] += jnp.sum(flat, axis=0, keepdims=True)


def _p2b_body(K, Q, nb_ref, ctr_ref, dist_ref, feat_ref, w1_ref, b1_ref,
              w2_ref, b2_ref, sw1_ref, sw2_ref, m1w_ref, m1b_ref, o1g_ref,
              o1f_ref, o1b_ref, shw_ref, shb_ref,
              g2_ref, y1_ref, ys_ref, st1_ref, sts_ref):
    b = pl.program_id(0)
    qi = pl.program_id(1)
    flat = _build_cat(K, Q, ctr_ref[0], nb_ref[0], dist_ref[0])
    enc1 = jnp.maximum(_dot(flat, w1_ref[...]) + b1_ref[...], 0.0)
    enc2 = jnp.maximum(_dot(flat, w2_ref[...]) + b2_ref[...], 0.0)

    def pool(enc, sw):
        a = _dot(enc, sw)
        ar = jnp.reshape(a, (Q, K, 64))
        er = jnp.reshape(enc, (Q, K, 64))
        mx = jnp.max(ar, axis=1, keepdims=True)
        ex = jnp.exp(ar - mx)
        s = ex / jnp.sum(ex, axis=1, keepdims=True)
        return jnp.sum(s * er, axis=1)

    g1 = pool(enc1, sw1_ref[...])
    g2 = pool(enc2, sw2_ref[...])
    g2_ref[0] = g2
    f = feat_ref[0]
    x0 = _dot(f, m1w_ref[...]) + m1b_ref[...]
    x0 = jnp.where(x0 >= 0, x0, 0.2 * x0)
    y1 = _dot(g1, o1g_ref[...]) + _dot(x0, o1f_ref[...]) + o1b_ref[...]
    y1_ref[0] = y1
    ys = _dot(f, shw_ref[...]) + shb_ref[...]
    ys_ref[0] = ys

    @pl.when(jnp.logical_and(b == 0, qi == 0))
    def _():
        st1_ref[...] = jnp.zeros_like(st1_ref)
        sts_ref[...] = jnp.zeros_like(sts_ref)

    st1_ref[0:1, :] += jnp.sum(y1, axis=0, keepdims=True)
    st1_ref[1:2, :] += jnp.sum(y1 * y1, axis=0, keepdims=True)
    sts_ref[0:1, :] += jnp.sum(ys, axis=0, keepdims=True)
    sts_ref[1:2, :] += jnp.sum(ys * ys, axis=0, keepdims=True)


def _p3_body(y1_ref, g2_ref, o2g_ref, o2f_ref, o2b_ref, sc1_ref, sh1_ref,
             y2_ref, st2_ref):
    b = pl.program_id(0)
    qi = pl.program_id(1)
    x1 = jnp.maximum(y1_ref[0] * sc1_ref[...] + sh1_ref[...], 0.0)
    y2 = _dot(g2_ref[0], o2g_ref[...]) + _dot(x1, o2f_ref[...]) + o2b_ref[...]
    y2_ref[0] = y2

    @pl.when(jnp.logical_and(b == 0, qi == 0))
    def _():
        st2_ref[...] = jnp.zeros_like(st2_ref)

    st2_ref[0:1, :] += jnp.sum(y2, axis=0, keepdims=True)
    st2_ref[1:2, :] += jnp.sum(y2 * y2, axis=0, keepdims=True)


def _p4_body(y2_ref, ys_ref, m2w_ref, m2b_ref, sc2_ref, sh2_ref, scs_ref,
             shs_ref, out_ref):
    x2 = jnp.maximum(y2_ref[0] * sc2_ref[...] + sh2_ref[...], 0.0)
    o = _dot(x2, m2w_ref[...]) + m2b_ref[...] + ys_ref[0] * scs_ref[...] \
        + shs_ref[...]
    out_ref[0] = jnp.where(o >= 0, o, 0.01 * o)


def kernel(coords, features, mlp1_w, mlp1_b, lse1_w, lse1_b, lse1_g, lse1_bt,
           pool1_sw, pool1_sb, pool1_ow, pool1_ob, pool1_g, pool1_bt,
           lse2_w, lse2_b, lse2_g, lse2_bt, pool2_sw, pool2_sb, pool2_ow,
           pool2_ob, pool2_g, pool2_bt, mlp2_w, mlp2_b, short_w, short_b,
           short_g, short_bt):
    B, N, _ = coords.shape
    K = 16
    h = mlp1_w.shape[0]
    d_in = mlp1_w.shape[1]
    d_out = pool1_sw.shape[0]
    d_fin = mlp2_w.shape[0]
    eps = 1e-6

    coords_pad = jnp.concatenate(
        [coords, jnp.zeros((B, N, 8 - coords.shape[2]), F32)], axis=2)
    coords_t = jnp.transpose(coords_pad, (0, 2, 1))
    feat_t = jnp.transpose(features[:, :, :, 0], (0, 2, 1))

    Q1 = 512
    nb1 = N // Q1
    p1 = pl.pallas_call(
        functools.partial(_p1_body, K, Q1, N),
        grid=(B, nb1),
        in_specs=[
            pl.BlockSpec((1, N, 8), lambda b, q: (b, 0, 0)),
            pl.BlockSpec((1, 8, N), lambda b, q: (b, 0, 0)),
        ],
        out_specs=[
            pl.BlockSpec((1, Q1, K), lambda b, q: (b, q, 0)),
            pl.BlockSpec((1, Q1, K), lambda b, q: (b, q, 0)),
        ],
        out_shape=[
            jax.ShapeDtypeStruct((B, N, K), jnp.int32),
            jax.ShapeDtypeStruct((B, N, K), F32),
        ],
        scratch_shapes=[pltpu.VMEM((Q1, N), F32)],
    )
    idx, dist = p1(coords_pad, coords_t)

    table = jnp.reshape(coords_pad, (B * N * 8,))
    gfn = _make_sc_gather(B * N, B * N * K)
    nb_flat = gfn(table, jnp.reshape(idx, (B * N * K,)))
    nbg = jnp.reshape(nb_flat, (B, N, K, 8))

    Q2 = 512
    nb2 = N // Q2
    cst = lambda b, q: (0, 0)
    p2a = pl.pallas_call(
        functools.partial(_p2a_body, K, Q2),
        grid=(B, nb2),
        in_specs=[
            pl.BlockSpec((1, Q2, K, 8), lambda b, q: (b, q, 0, 0)),
            pl.BlockSpec((1, Q2, 8), lambda b, q: (b, q, 0)),
            pl.BlockSpec((1, Q2, K), lambda b, q: (b, q, 0)),
        ],
        out_specs=[pl.BlockSpec((24, 16), cst)],
        out_shape=[jax.ShapeDtypeStruct((24, 16), F32)],
    )
    (mom,) = p2a(nbg, coords_pad, dist)

    M = B * N * K
    mu = mom[16, 0:10] / M
    m2 = mom[0:10, 0:10] / M
    cc = m2 - jnp.outer(mu, mu)

    def fold(w, bb, g, bt):
        mean = w @ mu + bb
        var = jnp.sum((w @ cc) * w, axis=1)
        sc = g / jnp.sqrt(var + eps)
        weff = jnp.zeros((16, h), F32).at[0:10, :].set((w * sc[:, None]).T)
        beff = (bb - mean) * sc + bt
        return weff, beff[None, :]

    w1eff, b1eff = fold(lse1_w, lse1_b, lse1_g, lse1_bt)
    w2eff, b2eff = fold(lse2_w, lse2_b, lse2_g, lse2_bt)

    p2b = pl.pallas_call(
        functools.partial(_p2b_body, K, Q2),
        grid=(B, nb2),
        in_specs=[
            pl.BlockSpec((1, Q2, K, 8), lambda b, q: (b, q, 0, 0)),
            pl.BlockSpec((1, Q2, 8), lambda b, q: (b, q, 0)),
            pl.BlockSpec((1, Q2, K), lambda b, q: (b, q, 0)),
            pl.BlockSpec((1, Q2, d_in), lambda b, q: (b, q, 0)),
            pl.BlockSpec((16, h), cst), pl.BlockSpec((1, h), cst),
            pl.BlockSpec((16, h), cst), pl.BlockSpec((1, h), cst),
            pl.BlockSpec((h, h), cst), pl.BlockSpec((h, h), cst),
            pl.BlockSpec((d_in, h), cst), pl.BlockSpec((1, h), cst),
            pl.BlockSpec((h, h), cst), pl.BlockSpec((h, h), cst),
            pl.BlockSpec((1, h), cst),
            pl.BlockSpec((d_in, d_fin), cst), pl.BlockSpec((1, d_fin), cst),
        ],
        out_specs=[
            pl.BlockSpec((1, Q2, h), lambda b, q: (b, q, 0)),
            pl.BlockSpec((1, Q2, h), lambda b, q: (b, q, 0)),
            pl.BlockSpec((1, Q2, d_fin), lambda b, q: (b, q, 0)),
            pl.BlockSpec((8, h), cst),
            pl.BlockSpec((8, d_fin), cst),
        ],
        out_shape=[
            jax.ShapeDtypeStruct((B, N, h), F32),
            jax.ShapeDtypeStruct((B, N, h), F32),
            jax.ShapeDtypeStruct((B, N, d_fin), F32),
            jax.ShapeDtypeStruct((8, h), F32),
            jax.ShapeDtypeStruct((8, d_fin), F32),
        ],
    )
    g2, y1, ys, st1, sts = p2b(
        nbg, coords_pad, dist, feat_t,
        w1eff, b1eff, w2eff, b2eff,
        pool1_sw[0:h, 0:h].T, pool2_sw[0:h, 0:h].T,
        mlp1_w.T, mlp1_b[None, :],
        pool1_ow[:, 0:h].T, pool1_ow[:, h:].T, pool1_ob[None, :],
        short_w.T, short_b[None, :])

    def bnstats(st, g, bt, cnt):
        mean = st[0] / cnt
        var = st[1] / cnt - mean * mean
        sc = g / jnp.sqrt(var + eps)
        return sc[None, :], (bt - mean * sc)[None, :]

    sc1, sh1 = bnstats(st1, pool1_g, pool1_bt, B * N)
    scs, shs = bnstats(sts, short_g, short_bt, B * N)

    Q3 = 2048
    nb3 = N // Q3
    p3 = pl.pallas_call(
        _p3_body,
        grid=(B, nb3),
        in_specs=[
            pl.BlockSpec((1, Q3, h), lambda b, q: (b, q, 0)),
            pl.BlockSpec((1, Q3, h), lambda b, q: (b, q, 0)),
            pl.BlockSpec((h, d_out), cst), pl.BlockSpec((h, d_out), cst),
            pl.BlockSpec((1, d_out), cst),
            pl.BlockSpec((1, h), cst), pl.BlockSpec((1, h), cst),
        ],
        out_specs=[
            pl.BlockSpec((1, Q3, d_out), lambda b, q: (b, q, 0)),
            pl.BlockSpec((8, d_out), cst),
        ],
        out_shape=[
            jax.ShapeDtypeStruct((B, N, d_out), F32),
            jax.ShapeDtypeStruct((8, d_out), F32),
        ],
    )
    y2, st2 = p3(y1, g2, pool2_ow[:, 0:h].T, pool2_ow[:, h:].T,
                 pool2_ob[None, :], sc1, sh1)

    sc2, sh2 = bnstats(st2, pool2_g, pool2_bt, B * N)

    p4 = pl.pallas_call(
        _p4_body,
        grid=(B, nb3),
        in_specs=[
            pl.BlockSpec((1, Q3, d_out), lambda b, q: (b, q, 0)),
            pl.BlockSpec((1, Q3, d_fin), lambda b, q: (b, q, 0)),
            pl.BlockSpec((d_out, d_fin), cst), pl.BlockSpec((1, d_fin), cst),
            pl.BlockSpec((1, d_out), cst), pl.BlockSpec((1, d_out), cst),
            pl.BlockSpec((1, d_fin), cst), pl.BlockSpec((1, d_fin), cst),
        ],
        out_specs=[pl.BlockSpec((1, Q3, d_fin), lambda b, q: (b, q, 0))],
        out_shape=[jax.ShapeDtypeStruct((B, N, d_fin), F32)],
    )
    (out,) = p4(y2, ys, mlp2_w.T, mlp2_b[None, :], sc2, sh2, scs, shs)

    return jnp.transpose(out, (0, 2, 1))[:, :, :, None]


# speedup vs baseline: 1.6284x; 1.0001x over previous
import functools

import jax
import jax.numpy as jnp
from jax import lax
from jax.experimental import pallas as pl
from jax.experimental.pallas import tpu as pltpu
from jax.experimental.pallas import tpu_sc as plsc

F32 = jnp.float32
BF16 = jnp.bfloat16
BIG = 3.0e38


def _dot(a, b):
    return jax.lax.dot_general(a, b, (((1,), (0,)), ((), ())),
                               preferred_element_type=F32,
                               precision=jax.lax.Precision.DEFAULT)


def _dott(a, b):
    return jax.lax.dot_general(a, b, (((0,), (0,)), ((), ())),
                               preferred_element_type=F32,
                               precision=jax.lax.Precision.HIGHEST)


def _p1_body(K, Q, N, keys_ref, kt_ref, idx_ref, dist_ref):
    b = pl.program_id(0)
    qi = pl.program_id(1)
    kt = kt_ref[0]
    q = keys_ref[0, pl.ds(qi * Q, Q), :]
    sqq = jnp.sum(q * q, axis=1, keepdims=True)
    sqk = jnp.sum(kt * kt, axis=0, keepdims=True)
    gram = jax.lax.dot_general(q, kt, (((1,), (0,)), ((), ())),
                               preferred_element_type=F32,
                               precision=jax.lax.Precision.DEFAULT)
    d2v = jnp.maximum(sqq + sqk - 2.0 * gram, 0.0)
    iota = jax.lax.broadcasted_iota(jnp.int32, (Q, N), 1)

    m = jnp.min(d2v, axis=1, keepdims=True)
    for k in range(K):
        sel = jnp.min(jnp.where(d2v == m, iota, N), axis=1,
                      keepdims=True)
        idx_ref[0, :, k:k + 1] = sel + b * N
        dist_ref[0, :, k:k + 1] = jnp.sqrt(m + 1e-12)
        if k < K - 1:
            d2v = jnp.where(iota == sel, BIG, d2v)
            m = jnp.min(d2v, axis=1, keepdims=True)


def _make_sc_gather(n_rows, n_idx):
    info = plsc.get_sparse_core_info()
    nc, ns = info.num_cores, info.num_subcores
    nw = nc * ns
    per_w = n_idx // nw
    mesh = plsc.VectorSubcoreMesh(core_axis_name="c", subcore_axis_name="s")

    @functools.partial(
        pl.kernel, mesh=mesh,
        out_type=jax.ShapeDtypeStruct((n_idx * 8,), F32),
        compiler_params=pltpu.CompilerParams(needs_layout_passes=False),
        scratch_types=[
            pltpu.VMEM((n_rows * 8,), F32),
            pltpu.VMEM((per_w,), jnp.int32),
            pltpu.VMEM((per_w * 8,), F32),
        ],
    )
    def g(table_hbm, idx_hbm, out_hbm, table_v, idx_v, out_v):
        wid = lax.axis_index("s") * nc + lax.axis_index("c")
        base = wid * per_w
        pltpu.sync_copy(table_hbm, table_v)
        pltpu.sync_copy(idx_hbm.at[pl.ds(base, per_w)], idx_v)
        lane = lax.iota(jnp.int32, 16)

        def body(gi, carry):
            idx16 = idx_v[pl.ds(gi * 16, 16)]
            rowbase = idx16 * 8
            obase = lane * 8 + gi * 128
            for c in range(3):
                vals = plsc.load_gather(table_v, [rowbase + c])
                plsc.store_scatter(out_v, [obase + c], vals)
            return carry

        jax.lax.fori_loop(0, per_w // 16, body, 0)
        pltpu.sync_copy(out_v, out_hbm.at[pl.ds(base * 8, per_w * 8)])

    return g


def _build_cat(K, Q, ctr, nbv, dd):
    c3 = jnp.broadcast_to(jnp.reshape(ctr[:, 0:3], (Q, 1, 3)), (Q, K, 3))
    nb3 = nbv[:, :, 0:3]
    cat = jnp.concatenate(
        [c3, nb3, c3 - nb3, jnp.reshape(dd, (Q, K, 1)),
         jnp.zeros((Q, K, 6), F32)], axis=2)
    return jnp.reshape(cat, (Q * K, 16))


def _p2a_body(K, Q, nb_ref, ctr_ref, dist_ref, mom_ref):
    b = pl.program_id(0)
    qi = pl.program_id(1)
    flat = _build_cat(K, Q, ctr_ref[0], nb_ref[0], dist_ref[0])

    @pl.when(jnp.logical_and(b == 0, qi == 0))
    def _():
        mom_ref[...] = jnp.zeros_like(mom_ref)

    mom_ref[0:16, :] += _dott(flat, flat)
    mom_ref[16:17, :] += jnp.sum(flat, axis=0, keepdims=True)


def _p2b_body(K, Q, nb_ref, ctr_ref, dist_ref, feat_ref, w1_ref, b1_ref,
              w2_ref, b2_ref, sw1_ref, sw2_ref, m1w_ref, m1b_ref, o1g_ref,
              o1f_ref, o1b_ref, shw_ref, shb_ref,
              g2_ref, y1_ref, ys_ref, st1_ref, sts_ref):
    b = pl.program_id(0)
    qi = pl.program_id(1)
    flat = _build_cat(K, Q, ctr_ref[0], nb_ref[0], dist_ref[0])
    enc1 = jnp.maximum(_dot(flat, w1_ref[...]) + b1_ref[...], 0.0)
    enc2 = jnp.maximum(_dot(flat, w2_ref[...]) + b2_ref[...], 0.0)

    def pool(enc, sw):
        a = _dot(enc, sw)
        ar = jnp.reshape(a, (Q, K, 64))
        er = jnp.reshape(enc, (Q, K, 64))
        mx = jnp.max(ar, axis=1, keepdims=True)
        ex = jnp.exp(ar - mx)
        s = ex / jnp.sum(ex, axis=1, keepdims=True)
        return jnp.sum(s * er, axis=1)

    g1 = pool(enc1, sw1_ref[...])
    g2 = pool(enc2, sw2_ref[...])
    g2_ref[0] = g2
    f = feat_ref[0]
    x0 = _dot(f, m1w_ref[...]) + m1b_ref[...]
    x0 = jnp.where(x0 >= 0, x0, 0.2 * x0)
    y1 = _dot(g1, o1g_ref[...]) + _dot(x0, o1f_ref[...]) + o1b_ref[...]
    y1_ref[0] = y1
    ys = _dot(f, shw_ref[...]) + shb_ref[...]
    ys_ref[0] = ys

    @pl.when(jnp.logical_and(b == 0, qi == 0))
    def _():
        st1_ref[...] = jnp.zeros_like(st1_ref)
        sts_ref[...] = jnp.zeros_like(sts_ref)

    st1_ref[0:1, :] += jnp.sum(y1, axis=0, keepdims=True)
    st1_ref[1:2, :] += jnp.sum(y1 * y1, axis=0, keepdims=True)
    sts_ref[0:1, :] += jnp.sum(ys, axis=0, keepdims=True)
    sts_ref[1:2, :] += jnp.sum(ys * ys, axis=0, keepdims=True)


def _p3_body(y1_ref, g2_ref, o2g_ref, o2f_ref, o2b_ref, sc1_ref, sh1_ref,
             y2_ref, st2_ref):
    b = pl.program_id(0)
    qi = pl.program_id(1)
    x1 = jnp.maximum(y1_ref[0] * sc1_ref[...] + sh1_ref[...], 0.0)
    y2 = _dot(g2_ref[0], o2g_ref[...]) + _dot(x1, o2f_ref[...]) + o2b_ref[...]
    y2_ref[0] = y2

    @pl.when(jnp.logical_and(b == 0, qi == 0))
    def _():
        st2_ref[...] = jnp.zeros_like(st2_ref)

    st2_ref[0:1, :] += jnp.sum(y2, axis=0, keepdims=True)
    st2_ref[1:2, :] += jnp.sum(y2 * y2, axis=0, keepdims=True)


def _p4_body(y2_ref, ys_ref, m2w_ref, m2b_ref, sc2_ref, sh2_ref, scs_ref,
             shs_ref, out_ref):
    x2 = jnp.maximum(y2_ref[0] * sc2_ref[...] + sh2_ref[...], 0.0)
    o = _dot(x2, m2w_ref[...]) + m2b_ref[...] + ys_ref[0] * scs_ref[...] \
        + shs_ref[...]
    out_ref[0] = jnp.where(o >= 0, o, 0.01 * o)


def kernel(coords, features, mlp1_w, mlp1_b, lse1_w, lse1_b, lse1_g, lse1_bt,
           pool1_sw, pool1_sb, pool1_ow, pool1_ob, pool1_g, pool1_bt,
           lse2_w, lse2_b, lse2_g, lse2_bt, pool2_sw, pool2_sb, pool2_ow,
           pool2_ob, pool2_g, pool2_bt, mlp2_w, mlp2_b, short_w, short_b,
           short_g, short_bt):
    B, N, _ = coords.shape
    K = 16
    h = mlp1_w.shape[0]
    d_in = mlp1_w.shape[1]
    d_out = pool1_sw.shape[0]
    d_fin = mlp2_w.shape[0]
    eps = 1e-6

    coords_pad = jnp.concatenate(
        [coords, jnp.zeros((B, N, 8 - coords.shape[2]), F32)], axis=2)
    coords_t = jnp.transpose(coords_pad, (0, 2, 1))
    feat_t = jnp.transpose(features[:, :, :, 0], (0, 2, 1))

    Q1 = 512
    nb1 = N // Q1
    p1 = pl.pallas_call(
        functools.partial(_p1_body, K, Q1, N),
        grid=(B, nb1),
        in_specs=[
            pl.BlockSpec((1, N, 8), lambda b, q: (b, 0, 0)),
            pl.BlockSpec((1, 8, N), lambda b, q: (b, 0, 0)),
        ],
        out_specs=[
            pl.BlockSpec((1, Q1, K), lambda b, q: (b, q, 0)),
            pl.BlockSpec((1, Q1, K), lambda b, q: (b, q, 0)),
        ],
        out_shape=[
            jax.ShapeDtypeStruct((B, N, K), jnp.int32),
            jax.ShapeDtypeStruct((B, N, K), F32),
        ],
    )
    idx, dist = p1(coords_pad, coords_t)

    table = jnp.reshape(coords_pad, (B * N * 8,))
    gfn = _make_sc_gather(B * N, B * N * K)
    nb_flat = gfn(table, jnp.reshape(idx, (B * N * K,)))
    nbg = jnp.reshape(nb_flat, (B, N, K, 8))

    Q2 = 512
    nb2 = N // Q2
    cst = lambda b, q: (0, 0)
    p2a = pl.pallas_call(
        functools.partial(_p2a_body, K, Q2),
        grid=(B, nb2),
        in_specs=[
            pl.BlockSpec((1, Q2, K, 8), lambda b, q: (b, q, 0, 0)),
            pl.BlockSpec((1, Q2, 8), lambda b, q: (b, q, 0)),
            pl.BlockSpec((1, Q2, K), lambda b, q: (b, q, 0)),
        ],
        out_specs=[pl.BlockSpec((24, 16), cst)],
        out_shape=[jax.ShapeDtypeStruct((24, 16), F32)],
    )
    (mom,) = p2a(nbg, coords_pad, dist)

    M = B * N * K
    mu = mom[16, 0:10] / M
    m2 = mom[0:10, 0:10] / M
    cc = m2 - jnp.outer(mu, mu)

    def fold(w, bb, g, bt):
        mean = w @ mu + bb
        var = jnp.sum((w @ cc) * w, axis=1)
        sc = g / jnp.sqrt(var + eps)
        weff = jnp.zeros((16, h), F32).at[0:10, :].set((w * sc[:, None]).T)
        beff = (bb - mean) * sc + bt
        return weff, beff[None, :]

    w1eff, b1eff = fold(lse1_w, lse1_b, lse1_g, lse1_bt)
    w2eff, b2eff = fold(lse2_w, lse2_b, lse2_g, lse2_bt)

    p2b = pl.pallas_call(
        functools.partial(_p2b_body, K, Q2),
        grid=(B, nb2),
        in_specs=[
            pl.BlockSpec((1, Q2, K, 8), lambda b, q: (b, q, 0, 0)),
            pl.BlockSpec((1, Q2, 8), lambda b, q: (b, q, 0)),
            pl.BlockSpec((1, Q2, K), lambda b, q: (b, q, 0)),
            pl.BlockSpec((1, Q2, d_in), lambda b, q: (b, q, 0)),
            pl.BlockSpec((16, h), cst), pl.BlockSpec((1, h), cst),
            pl.BlockSpec((16, h), cst), pl.BlockSpec((1, h), cst),
            pl.BlockSpec((h, h), cst), pl.BlockSpec((h, h), cst),
            pl.BlockSpec((d_in, h), cst), pl.BlockSpec((1, h), cst),
            pl.BlockSpec((h, h), cst), pl.BlockSpec((h, h), cst),
            pl.BlockSpec((1, h), cst),
            pl.BlockSpec((d_in, d_fin), cst), pl.BlockSpec((1, d_fin), cst),
        ],
        out_specs=[
            pl.BlockSpec((1, Q2, h), lambda b, q: (b, q, 0)),
            pl.BlockSpec((1, Q2, h), lambda b, q: (b, q, 0)),
            pl.BlockSpec((1, Q2, d_fin), lambda b, q: (b, q, 0)),
            pl.BlockSpec((8, h), cst),
            pl.BlockSpec((8, d_fin), cst),
        ],
        out_shape=[
            jax.ShapeDtypeStruct((B, N, h), F32),
            jax.ShapeDtypeStruct((B, N, h), F32),
            jax.ShapeDtypeStruct((B, N, d_fin), F32),
            jax.ShapeDtypeStruct((8, h), F32),
            jax.ShapeDtypeStruct((8, d_fin), F32),
        ],
    )
    g2, y1, ys, st1, sts = p2b(
        nbg, coords_pad, dist, feat_t,
        w1eff, b1eff, w2eff, b2eff,
        pool1_sw[0:h, 0:h].T, pool2_sw[0:h, 0:h].T,
        mlp1_w.T, mlp1_b[None, :],
        pool1_ow[:, 0:h].T, pool1_ow[:, h:].T, pool1_ob[None, :],
        short_w.T, short_b[None, :])

    def bnstats(st, g, bt, cnt):
        mean = st[0] / cnt
        var = st[1] / cnt - mean * mean
        sc = g / jnp.sqrt(var + eps)
        return sc[None, :], (bt - mean * sc)[None, :]

    sc1, sh1 = bnstats(st1, pool1_g, pool1_bt, B * N)
    scs, shs = bnstats(sts, short_g, short_bt, B * N)

    Q3 = 2048
    nb3 = N // Q3
    p3 = pl.pallas_call(
        _p3_body,
        grid=(B, nb3),
        in_specs=[
            pl.BlockSpec((1, Q3, h), lambda b, q: (b, q, 0)),
            pl.BlockSpec((1, Q3, h), lambda b, q: (b, q, 0)),
            pl.BlockSpec((h, d_out), cst), pl.BlockSpec((h, d_out), cst),
            pl.BlockSpec((1, d_out), cst),
            pl.BlockSpec((1, h), cst), pl.BlockSpec((1, h), cst),
        ],
        out_specs=[
            pl.BlockSpec((1, Q3, d_out), lambda b, q: (b, q, 0)),
            pl.BlockSpec((8, d_out), cst),
        ],
        out_shape=[
            jax.ShapeDtypeStruct((B, N, d_out), F32),
            jax.ShapeDtypeStruct((8, d_out), F32),
        ],
    )
    y2, st2 = p3(y1, g2, pool2_ow[:, 0:h].T, pool2_ow[:, h:].T,
                 pool2_ob[None, :], sc1, sh1)

    sc2, sh2 = bnstats(st2, pool2_g, pool2_bt, B * N)

    p4 = pl.pallas_call(
        _p4_body,
        grid=(B, nb3),
        in_specs=[
            pl.BlockSpec((1, Q3, d_out), lambda b, q: (b, q, 0)),
            pl.BlockSpec((1, Q3, d_fin), lambda b, q: (b, q, 0)),
            pl.BlockSpec((d_out, d_fin), cst), pl.BlockSpec((1, d_fin), cst),
            pl.BlockSpec((1, d_out), cst), pl.BlockSpec((1, d_out), cst),
            pl.BlockSpec((1, d_fin), cst), pl.BlockSpec((1, d_fin), cst),
        ],
        out_specs=[pl.BlockSpec((1, Q3, d_fin), lambda b, q: (b, q, 0))],
        out_shape=[jax.ShapeDtypeStruct((B, N, d_fin), F32)],
    )
    (out,) = p4(y2, ys, mlp2_w.T, mlp2_b[None, :], sc2, sh2, scs, shs)

    return jnp.transpose(out, (0, 2, 1))[:, :, :, None]
